# baseline (speedup 1.0000x reference)
import functools

import jax
import jax.numpy as jnp
from jax import lax
from jax.experimental import pallas as pl
from jax.experimental.pallas import tpu as pltpu
from jax.experimental.pallas import tpu_sc as plsc


@functools.cache
def _build_fused(seq: int, batch: int, dim: int, nv: int):
    info = plsc.get_sparse_core_info()
    nc, ns = info.num_cores, info.num_subcores
    npos = seq * batch
    ppt = npos // ns
    assert ppt * ns == npos
    unit = batch // 2
    upt = ppt // unit
    assert upt * unit == ppt and ppt % 8 == 0
    dpc = dim // nc
    assert dpc * nc == dim

    mesh = plsc.VectorSubcoreMesh(core_axis_name="c", subcore_axis_name="s")

    @functools.partial(
        pl.kernel,
        out_type=jax.ShapeDtypeStruct((seq * dim * batch,), jnp.float32),
        mesh=mesh,
        scratch_types=[
            pltpu.VMEM((ppt,), jnp.int32),
            [pltpu.VMEM((unit,), jnp.float32) for _ in range(4)],
            pltpu.VMEM_SHARED((nv,), jnp.float32),
            pltpu.SemaphoreType.DMA,
            [pltpu.SemaphoreType.DMA for _ in range(4)],
            [pltpu.SemaphoreType.DMA for _ in range(4)],
        ],
        compiler_params=pltpu.CompilerParams(use_tc_tiling_on_sc=True),
    )
    def fused_kernel(table_hbm, idx_hbm, out_hbm, idx_v, gath_v, row_sh,
                     ssem, gsem, osem):
        cid = lax.axis_index("c")
        tid = lax.axis_index("s")
        p0 = tid * ppt
        d0 = cid * dpc

        def _idx_args(k):
            p = p0 + k * unit
            return (
                idx_hbm.at[p // batch, pl.ds(p % batch, unit)],
                idx_v.at[pl.ds(k * unit, unit)],
                gsem[0],
            )

        for k in range(upt):
            pltpu.async_copy(*_idx_args(k))
        for k in range(upt):
            pltpu.make_async_copy(*_idx_args(k)).wait()

        @pl.when(tid == 0)
        def _stage_first():
            pltpu.async_copy(table_hbm.at[d0], row_sh, ssem)

        @pl.loop(0, dpc)
        def _dim(dl):
            dg = d0 + dl

            @pl.when(tid == 0)
            def _wait_stage():
                pltpu.make_async_copy(table_hbm.at[dg], row_sh, ssem).wait()

            plsc.subcore_barrier()

            def _gather_args(k, b):
                return (
                    row_sh.at[idx_v.at[pl.ds(k * unit, unit)]],
                    gath_v[b],
                    gsem[b],
                )

            def _write_args(k, b):
                p = p0 + k * unit
                off = (p // batch * dim + dg) * batch + p % batch
                return (gath_v[b], out_hbm.at[pl.ds(off, unit)], osem[b])

            nb = 4
            for k in range(upt):
                b = k % nb
                if k == 0:
                    for j in range(min(nb - 1, upt)):
                        pltpu.async_copy(*_gather_args(j, j % nb))
                if k + nb - 1 < upt:
                    if k >= 1:
                        pltpu.make_async_copy(*_write_args(k - 1, (k - 1) % nb)).wait()
                    pltpu.async_copy(*_gather_args(k + nb - 1, (k + nb - 1) % nb))
                pltpu.make_async_copy(*_gather_args(k, b)).wait()
                pltpu.async_copy(*_write_args(k, b))

            plsc.subcore_barrier()

            @pl.when(jnp.logical_and(tid == 0, dl + 1 < dpc))
            def _stage_next():
                pltpu.async_copy(table_hbm.at[dg + 1], row_sh, ssem)

            for k in range(max(0, upt - nb), upt):
                pltpu.make_async_copy(*_write_args(k, k % nb)).wait()

    return fused_kernel


def kernel(token_ids, table):
    b, s = token_ids.shape
    nv, dim = table.shape
    table_t = table.T
    idx_t = token_ids.T.astype(jnp.int32)
    out_flat = _build_fused(s, b, dim, nv)(table_t, idx_t)
    z = lax.optimization_barrier(jnp.float32(0.0))
    out_t = out_flat.reshape(s, dim, b) + z
    return jnp.transpose(out_t, (2, 0, 1))


# speedup vs baseline: 1.1486x; 1.1486x over previous
import functools

import jax
import jax.numpy as jnp
from jax import lax
from jax.experimental import pallas as pl
from jax.experimental.pallas import tpu as pltpu
from jax.experimental.pallas import tpu_sc as plsc


@functools.cache
def _build_fused(seq: int, batch: int, dim: int, nv: int):
    info = plsc.get_sparse_core_info()
    nc, ns = info.num_cores, info.num_subcores
    npos = seq * batch
    ppt = npos // ns
    assert ppt * ns == npos
    unit = batch // 2
    upt = ppt // unit
    assert upt * unit == ppt and ppt % 8 == 0
    dpc = dim // nc
    assert dpc * nc == dim

    mesh = plsc.VectorSubcoreMesh(core_axis_name="c", subcore_axis_name="s")

    @functools.partial(
        pl.kernel,
        out_type=jax.ShapeDtypeStruct((seq * dim * batch,), jnp.float32),
        mesh=mesh,
        scratch_types=[
            pltpu.VMEM((ppt,), jnp.int32),
            [pltpu.VMEM((unit,), jnp.float32) for _ in range(4)],
            pltpu.VMEM_SHARED((nv,), jnp.float32),
            pltpu.SemaphoreType.DMA,
            [pltpu.SemaphoreType.DMA for _ in range(4)],
            [pltpu.SemaphoreType.DMA for _ in range(4)],
        ],
        compiler_params=pltpu.CompilerParams(use_tc_tiling_on_sc=True),
    )
    def fused_kernel(table_hbm, idx_hbm, out_hbm, idx_v, gath_v, row_sh,
                     ssem, gsem, osem):
        cid = lax.axis_index("c")
        tid = lax.axis_index("s")
        p0 = tid * ppt
        d0 = cid * dpc

        def _idx_args(k):
            p = p0 + k * unit
            return (
                idx_hbm.at[p // batch, pl.ds(p % batch, unit)],
                idx_v.at[pl.ds(k * unit, unit)],
                gsem[0],
            )

        for k in range(upt):
            pltpu.async_copy(*_idx_args(k))
        for k in range(upt):
            pltpu.make_async_copy(*_idx_args(k)).wait()

        @pl.when(tid == 0)
        def _stage_first():
            pltpu.async_copy(table_hbm.at[d0], row_sh, ssem)

        @pl.loop(0, dpc)
        def _dim(dl):
            dg = d0 + dl

            @pl.when(tid == 0)
            def _wait_stage():
                pltpu.make_async_copy(table_hbm.at[dg], row_sh, ssem).wait()

            plsc.subcore_barrier()

            def _gather_args(k, b):
                return (
                    row_sh.at[idx_v.at[pl.ds(k * unit, unit)]],
                    gath_v[b],
                    gsem[b],
                )

            def _write_args(k, b):
                p = p0 + k * unit
                off = (p // batch * dim + dg) * batch + p % batch
                return (gath_v[b], out_hbm.at[pl.ds(off, unit)], osem[b])

            nb = 4
            for k in range(upt):
                b = k % nb
                if k == 0:
                    for j in range(min(nb - 1, upt)):
                        pltpu.async_copy(*_gather_args(j, j % nb))
                if k + nb - 1 < upt:
                    if k >= 1:
                        pltpu.make_async_copy(*_write_args(k - 1, (k - 1) % nb)).wait()
                    pltpu.async_copy(*_gather_args(k + nb - 1, (k + nb - 1) % nb))
                pltpu.make_async_copy(*_gather_args(k, b)).wait()
                pltpu.async_copy(*_write_args(k, b))

            plsc.subcore_barrier()

            @pl.when(jnp.logical_and(tid == 0, dl + 1 < dpc))
            def _stage_next():
                pltpu.async_copy(table_hbm.at[dg + 1], row_sh, ssem)

            for k in range(max(0, upt - nb), upt):
                pltpu.make_async_copy(*_write_args(k, k % nb)).wait()

    return fused_kernel


def kernel(token_ids, table):
    b, s = token_ids.shape
    nv, dim = table.shape
    table_t = table.T
    idx_t = token_ids.T.astype(jnp.int32)
    out_flat = _build_fused(s, b, dim, nv)(table_t, idx_t)
    out_t = out_flat.reshape(s, dim, b)
    return jnp.transpose(out_t, (2, 0, 1))
